# baseline (speedup 1.0000x reference)
import numpy as np
import jax
import jax.numpy as jnp
from jax.experimental import pallas as pl
from jax.experimental.pallas import tpu as pltpu

_NEG_SLOPE = 0.1


def _bilinear_matrix_np(in_size, out_size):
    if out_size == 1:
        src = np.zeros((1,), dtype=np.float64)
    else:
        src = np.arange(out_size, dtype=np.float64) * (in_size - 1) / (out_size - 1)
    i0 = np.clip(np.floor(src).astype(np.int64), 0, in_size - 1)
    i1 = np.clip(i0 + 1, 0, in_size - 1)
    w1 = src - i0
    mat = np.zeros((out_size, in_size), dtype=np.float32)
    mat[np.arange(out_size), i0] += (1.0 - w1).astype(np.float32)
    mat[np.arange(out_size), i1] += w1.astype(np.float32)
    return mat


def _pad_hw1(x):
    H, W, C = x.shape
    zr = jnp.zeros((1, W, C), x.dtype)
    x = jnp.concatenate([zr, x, zr], axis=0)
    zc = jnp.zeros((H + 2, 1, C), x.dtype)
    return jnp.concatenate([zc, x, zc], axis=1)


def _conv3x3_acc(x3, w_ref, acc):
    H, W, C = x3.shape
    xp = _pad_hw1(x3)
    for kh in range(3):
        row = xp[kh:kh + H]
        win = jnp.concatenate(
            [row[:, 0:W], row[:, 1:W + 1], row[:, 2:W + 2]], axis=-1)
        acc = acc + jnp.dot(win.reshape(H * W, 3 * C), w_ref[kh],
                            preferred_element_type=jnp.float32)
    return acc


def _fused_kernel(x_ref, ah_ref, bw_ref, w1_ref, b1_ref,
                  s_ref, wa_ref, wb_ref, b2_ref, o_ref):
    H2 = ah_ref.shape[0]
    Cout = w1_ref.shape[2]
    Cin = w1_ref.shape[1] // 3
    W2 = bw_ref.shape[1] // Cin

    x2d = x_ref[0]
    y = jnp.dot(ah_ref[...], x2d,
                preferred_element_type=jnp.float32)
    up = jnp.dot(y.astype(jnp.bfloat16), bw_ref[...],
                 preferred_element_type=jnp.float32)
    up3 = up.reshape(H2, W2, Cin).astype(jnp.bfloat16)

    acc1 = jnp.zeros((H2 * W2, Cout), jnp.float32)
    acc1 = _conv3x3_acc(up3, w1_ref, acc1)
    acc1 = acc1 + b1_ref[...]
    acc1 = jnp.where(acc1 >= 0.0, acc1, _NEG_SLOPE * acc1)
    a1 = acc1.astype(jnp.bfloat16).reshape(H2, W2, Cout)

    s_nhwc = s_ref[0]
    acc2 = jnp.zeros((H2 * W2, Cout), jnp.float32)
    acc2 = _conv3x3_acc(a1, wa_ref, acc2)
    acc2 = _conv3x3_acc(s_nhwc, wb_ref, acc2)
    acc2 = acc2 + b2_ref[...]
    acc2 = jnp.where(acc2 >= 0.0, acc2, _NEG_SLOPE * acc2)
    o_ref[0] = acc2.reshape(H2, W2, Cout)


def kernel(input_nchw, skip_nchw, w1r, b1_2, w2a, w2b, b2_2):
    N, Cin, H, W = input_nchw.shape
    H2, W2 = 2 * H, 2 * W
    Cout = w1r.shape[-1]

    ah = jnp.asarray(_bilinear_matrix_np(H, H2), jnp.bfloat16)
    awt = _bilinear_matrix_np(W, W2).T
    bw = jnp.asarray(np.kron(awt, np.eye(Cin, dtype=np.float32)),
                     jnp.bfloat16)

    w1 = w1r.astype(jnp.bfloat16)
    wa = w2a.astype(jnp.bfloat16)
    wb = w2b.astype(jnp.bfloat16)

    def run_chunk(x_chunk, skip_chunk):
        n = x_chunk.shape[0]
        x = jnp.transpose(x_chunk, (0, 2, 3, 1)).reshape(
            n, H, W * Cin).astype(jnp.bfloat16)
        skip = jnp.transpose(skip_chunk, (0, 2, 3, 1)).astype(jnp.bfloat16)
        out = pl.pallas_call(
            _fused_kernel,
            out_shape=jax.ShapeDtypeStruct((n, H2, W2, Cout), jnp.float32),
            grid=(n,),
            in_specs=[
                pl.BlockSpec((1, H, W * Cin), lambda i: (i, 0, 0)),
                pl.BlockSpec((H2, H), lambda i: (0, 0)),
                pl.BlockSpec((W * Cin, W2 * Cin), lambda i: (0, 0)),
                pl.BlockSpec((3, 3 * Cin, Cout), lambda i: (0, 0, 0)),
                pl.BlockSpec((1, Cout), lambda i: (0, 0)),
                pl.BlockSpec((1, H2, W2, Cout), lambda i: (i, 0, 0, 0)),
                pl.BlockSpec((3, 3 * Cout, Cout), lambda i: (0, 0, 0)),
                pl.BlockSpec((3, 3 * Cout, Cout), lambda i: (0, 0, 0)),
                pl.BlockSpec((1, Cout), lambda i: (0, 0)),
            ],
            out_specs=pl.BlockSpec((1, H2, W2, Cout), lambda i: (i, 0, 0, 0)),
            compiler_params=pltpu.CompilerParams(
                dimension_semantics=("parallel",)),
        )(x, ah, bw, w1, b1_2, skip, wa, wb, b2_2)
        return jnp.transpose(out, (0, 3, 1, 2))

    half = N // 2
    o0 = run_chunk(input_nchw[:half], skip_nchw[:half])
    o1 = run_chunk(input_nchw[half:], skip_nchw[half:])
    return jnp.concatenate([o0, o1], axis=0)


# speedup vs baseline: 1.1679x; 1.1679x over previous
import numpy as np
import jax
import jax.numpy as jnp
from jax.experimental import pallas as pl
from jax.experimental.pallas import tpu as pltpu

_NEG_SLOPE = 0.1


def _bilinear_matrix_np(in_size, out_size):
    if out_size == 1:
        src = np.zeros((1,), dtype=np.float64)
    else:
        src = np.arange(out_size, dtype=np.float64) * (in_size - 1) / (out_size - 1)
    i0 = np.clip(np.floor(src).astype(np.int64), 0, in_size - 1)
    i1 = np.clip(i0 + 1, 0, in_size - 1)
    w1 = src - i0
    mat = np.zeros((out_size, in_size), dtype=np.float32)
    mat[np.arange(out_size), i0] += (1.0 - w1).astype(np.float32)
    mat[np.arange(out_size), i1] += w1.astype(np.float32)
    return mat


def _pad_hw1(x):
    H, W, C = x.shape
    zr = jnp.zeros((1, W, C), x.dtype)
    x = jnp.concatenate([zr, x, zr], axis=0)
    zc = jnp.zeros((H + 2, 1, C), x.dtype)
    return jnp.concatenate([zc, x, zc], axis=1)


def _conv3x3_acc(x3, w_ref, acc):
    H, W, C = x3.shape
    xp = _pad_hw1(x3)
    for kh in range(3):
        row = xp[kh:kh + H]
        win = jnp.concatenate(
            [row[:, 0:W], row[:, 1:W + 1], row[:, 2:W + 2]], axis=-1)
        acc = acc + jnp.dot(win.reshape(H * W, 3 * C), w_ref[kh],
                            preferred_element_type=jnp.float32)
    return acc


def _fused_kernel(x_ref, ah_ref, bw_ref, w1_ref, b1_ref,
                  s_ref, wa_ref, wb_ref, b2_ref, o_ref):
    H2 = ah_ref.shape[0]
    Cout = w1_ref.shape[2]
    Cin = w1_ref.shape[1] // 3
    W2 = bw_ref.shape[1] // Cin

    for i in range(x_ref.shape[0]):
        x2d = x_ref[i]
        y = jnp.dot(ah_ref[...], x2d,
                    preferred_element_type=jnp.float32)
        up = jnp.dot(y.astype(jnp.bfloat16), bw_ref[...],
                     preferred_element_type=jnp.float32)
        up3 = up.reshape(H2, W2, Cin).astype(jnp.bfloat16)

        acc1 = jnp.zeros((H2 * W2, Cout), jnp.float32)
        acc1 = _conv3x3_acc(up3, w1_ref, acc1)
        acc1 = acc1 + b1_ref[...]
        acc1 = jnp.where(acc1 >= 0.0, acc1, _NEG_SLOPE * acc1)
        a1 = acc1.astype(jnp.bfloat16).reshape(H2, W2, Cout)

        acc2 = jnp.zeros((H2 * W2, Cout), jnp.float32)
        acc2 = _conv3x3_acc(a1, wa_ref, acc2)
        acc2 = _conv3x3_acc(s_ref[i], wb_ref, acc2)
        acc2 = acc2 + b2_ref[...]
        acc2 = jnp.where(acc2 >= 0.0, acc2, _NEG_SLOPE * acc2)
        o_ref[i] = acc2.reshape(H2, W2, Cout)


def kernel(input_nchw, skip_nchw, w1r, b1_2, w2a, w2b, b2_2):
    N, Cin, H, W = input_nchw.shape
    H2, W2 = 2 * H, 2 * W
    Cout = w1r.shape[-1]

    ah = jnp.asarray(_bilinear_matrix_np(H, H2), jnp.bfloat16)
    awt = _bilinear_matrix_np(W, W2).T
    bw = jnp.asarray(np.kron(awt, np.eye(Cin, dtype=np.float32)),
                     jnp.bfloat16)

    w1 = w1r.astype(jnp.bfloat16)
    wa = w2a.astype(jnp.bfloat16)
    wb = w2b.astype(jnp.bfloat16)

    x = jnp.transpose(input_nchw, (0, 2, 3, 1)).reshape(
        N, H, W * Cin).astype(jnp.bfloat16)
    skip = jnp.transpose(skip_nchw, (0, 2, 3, 1)).astype(jnp.bfloat16)

    out = pl.pallas_call(
        _fused_kernel,
        out_shape=jax.ShapeDtypeStruct((N, H2, W2, Cout), jnp.float32),
        grid=(N // 2,),
        in_specs=[
            pl.BlockSpec((2, H, W * Cin), lambda n: (n, 0, 0)),
            pl.BlockSpec((H2, H), lambda n: (0, 0)),
            pl.BlockSpec((W * Cin, W2 * Cin), lambda n: (0, 0)),
            pl.BlockSpec((3, 3 * Cin, Cout), lambda n: (0, 0, 0)),
            pl.BlockSpec((1, Cout), lambda n: (0, 0)),
            pl.BlockSpec((2, H2, W2, Cout), lambda n: (n, 0, 0, 0)),
            pl.BlockSpec((3, 3 * Cout, Cout), lambda n: (0, 0, 0)),
            pl.BlockSpec((3, 3 * Cout, Cout), lambda n: (0, 0, 0)),
            pl.BlockSpec((1, Cout), lambda n: (0, 0)),
        ],
        out_specs=pl.BlockSpec((2, H2, W2, Cout), lambda n: (n, 0, 0, 0)),
        compiler_params=pltpu.CompilerParams(
            dimension_semantics=("parallel",)),
    )(x, ah, bw, w1, b1_2, skip, wa, wb, b2_2)
    return jnp.transpose(out, (0, 3, 1, 2))


# speedup vs baseline: 1.2651x; 1.0832x over previous
import numpy as np
import jax
import jax.numpy as jnp
from jax.experimental import pallas as pl
from jax.experimental.pallas import tpu as pltpu

_NEG_SLOPE = 0.1


def _bilinear_matrix_np(in_size, out_size):
    if out_size == 1:
        src = np.zeros((1,), dtype=np.float64)
    else:
        src = np.arange(out_size, dtype=np.float64) * (in_size - 1) / (out_size - 1)
    i0 = np.clip(np.floor(src).astype(np.int64), 0, in_size - 1)
    i1 = np.clip(i0 + 1, 0, in_size - 1)
    w1 = src - i0
    mat = np.zeros((out_size, in_size), dtype=np.float32)
    mat[np.arange(out_size), i0] += (1.0 - w1).astype(np.float32)
    mat[np.arange(out_size), i1] += w1.astype(np.float32)
    return mat


def _pad_hw1(x):
    H, W, C = x.shape
    zr = jnp.zeros((1, W, C), x.dtype)
    x = jnp.concatenate([zr, x, zr], axis=0)
    zc = jnp.zeros((H + 2, 1, C), x.dtype)
    return jnp.concatenate([zc, x, zc], axis=1)


def _conv3x3_acc(x3, w_ref, acc):
    H, W, C = x3.shape
    xp = _pad_hw1(x3)
    for kh in range(3):
        row = xp[kh:kh + H]
        win = jnp.concatenate(
            [row[:, 0:W], row[:, 1:W + 1], row[:, 2:W + 2]], axis=-1)
        acc = acc + jnp.dot(win.reshape(H * W, 3 * C), w_ref[kh],
                            preferred_element_type=jnp.float32)
    return acc


def _fused_kernel(x_ref, ah_ref, bw_ref, w1_ref, b1_ref,
                  s_ref, wa_ref, wb_ref, b2_ref, o_ref):
    H2 = ah_ref.shape[0]
    Cout = w1_ref.shape[2]
    Cin = w1_ref.shape[1] // 3
    W2 = bw_ref.shape[1] // Cin

    for i in range(1):
        x2d = x_ref[i]
        y = jnp.dot(ah_ref[...], x2d,
                    preferred_element_type=jnp.float32)
        up = jnp.dot(y.astype(jnp.bfloat16), bw_ref[...],
                     preferred_element_type=jnp.float32)
        up3 = up.reshape(H2, W2, Cin).astype(jnp.bfloat16)

        acc1 = jnp.zeros((H2 * W2, Cout), jnp.float32)
        acc1 = _conv3x3_acc(up3, w1_ref, acc1)
        acc1 = acc1 + b1_ref[...]
        acc1 = jnp.where(acc1 >= 0.0, acc1, _NEG_SLOPE * acc1)
        a1 = acc1.astype(jnp.bfloat16).reshape(H2, W2, Cout)

        acc2 = jnp.zeros((H2 * W2, Cout), jnp.float32)
        acc2 = _conv3x3_acc(a1, wa_ref, acc2)
        acc2 = _conv3x3_acc(s_ref[i].astype(jnp.bfloat16), wb_ref, acc2)
        acc2 = acc2 + b2_ref[...]
        acc2 = jnp.where(acc2 >= 0.0, acc2, _NEG_SLOPE * acc2)
        o_ref[i] = acc2.reshape(H2, W2, Cout)


def kernel(input_nchw, skip_nchw, w1r, b1_2, w2a, w2b, b2_2):
    N, Cin, H, W = input_nchw.shape
    H2, W2 = 2 * H, 2 * W
    Cout = w1r.shape[-1]

    ah = jnp.asarray(_bilinear_matrix_np(H, H2), jnp.bfloat16)
    awt = _bilinear_matrix_np(W, W2).T
    bw = jnp.asarray(np.kron(awt, np.eye(Cin, dtype=np.float32)),
                     jnp.bfloat16)

    w1 = w1r.astype(jnp.bfloat16)
    wa = w2a.astype(jnp.bfloat16)
    wb = w2b.astype(jnp.bfloat16)

    x = jnp.transpose(input_nchw, (0, 2, 3, 1)).reshape(
        N, H, W * Cin).astype(jnp.bfloat16)
    skip = jnp.transpose(skip_nchw, (0, 2, 3, 1))

    out = pl.pallas_call(
        _fused_kernel,
        out_shape=jax.ShapeDtypeStruct((N, H2, W2, Cout), jnp.float32),
        grid=(N,),
        in_specs=[
            pl.BlockSpec((1, H, W * Cin), lambda n: (n, 0, 0)),
            pl.BlockSpec((H2, H), lambda n: (0, 0)),
            pl.BlockSpec((W * Cin, W2 * Cin), lambda n: (0, 0)),
            pl.BlockSpec((3, 3 * Cin, Cout), lambda n: (0, 0, 0)),
            pl.BlockSpec((1, Cout), lambda n: (0, 0)),
            pl.BlockSpec((1, H2, W2, Cout), lambda n: (n, 0, 0, 0)),
            pl.BlockSpec((3, 3 * Cout, Cout), lambda n: (0, 0, 0)),
            pl.BlockSpec((3, 3 * Cout, Cout), lambda n: (0, 0, 0)),
            pl.BlockSpec((1, Cout), lambda n: (0, 0)),
        ],
        out_specs=pl.BlockSpec((1, H2, W2, Cout), lambda n: (n, 0, 0, 0)),
        compiler_params=pltpu.CompilerParams(
            dimension_semantics=("parallel",)),
    )(x, ah, bw, w1, b1_2, skip, wa, wb, b2_2)
    return jnp.transpose(out, (0, 3, 1, 2))
